# baseline (speedup 1.0000x reference)
import jax
import jax.numpy as jnp
from jax import lax
from jax.experimental import pallas as pl
from jax.experimental.pallas import tpu as pltpu
from jax.experimental.pallas import tpu_sc as plsc

_N = 50000
_E = 800000
_K = 8
_C = 6256
_NPAD = _K * _C
_AROWS = 25088
_DUMMY = 4 * _C
_NT = 16
_ET = _E // _NT
_BE = 2000
_NB = _ET // _BE
_PEND = 2048
_BR = 256
_RPT = _AROWS // _NT
_DNT = 392

_NORM = 0.25
_RCUT = 5.0
_PREF = _NORM * 0.6324555320336759 * 0.5

_CC = (0.9999999998456892, -4.934802152338345, 4.058709712157277,
       -1.3352188829542484, 0.2349718054358449, -0.024456210213263066)
_CS = (3.141592653552295, -5.1677127683361075, 2.5501634533581243,
       -0.5992538712565725, 0.08205879138675946, -0.007042952766296561)


def _poly(w, coeffs):
    acc = jnp.float32(coeffs[-1])
    for c in reversed(coeffs[:-1]):
        acc = acc * w + jnp.float32(c)
    return acc


def _sc_body(tbl_hbm, eidx_hbm, pe_hbm, out_hbm,
             tbl, acc, cbat, nbat, pend_c, pend_n,
             crow, nrow, vx0, vx1, vx2, vx3, vx4, vx5, vx6, vx7,
             tstage, ostage, zbuf, pstage,
             sem1, sem2, sem3):
    cid = lax.axis_index("c")
    sid = lax.axis_index("s")
    iota = lax.iota(jnp.int32, 16)
    zf = jnp.zeros((16,), jnp.float32)
    zi = jnp.zeros((16,), jnp.int32)
    vxbufs = (vx0, vx1, vx2, vx3, vx4, vx5, vx6, vx7)

    tb = sid * 3128

    @pl.when(sid < _NT - 1)
    def _stage_main():
        pltpu.sync_copy(tbl_hbm.at[pl.ds(tb, 3128)], tbl.at[pl.ds(tb, 3128)])

    @pl.when(sid == _NT - 1)
    def _stage_tail():
        pltpu.sync_copy(tbl_hbm.at[pl.ds(15 * 3128, _N - 15 * 3128)],
                        tbl.at[pl.ds(15 * 3128, _N - 15 * 3128)])

    pltpu.sync_copy(pe_hbm, pstage)

    def zb_body(k, _):
        plsc.store_scatter(zbuf, [jnp.full((16,), k, jnp.int32), iota], zf)
        plsc.store_scatter(zbuf, [jnp.full((16,), k, jnp.int32), 16 + iota],
                           zf)
        plsc.store_scatter(zbuf, [jnp.full((16,), k, jnp.int32), 32 + iota],
                           zf)
        return 0
    lax.fori_loop(0, 16, zb_body, 0)

    def zp_body(k, _):
        plsc.store_scatter(pend_c, [k * 16 + iota], zi)
        plsc.store_scatter(pend_n, [k * 16 + iota], zi)
        return 0
    lax.fori_loop(0, _PEND // 16, zp_body, 0)

    for pidx in range(_K // 2):
        chunk = cid * (_K // 2) + pidx
        c_lo = chunk * _C
        abase = sid * _RPT

        for zrep in range(_RPT // 16):
            pltpu.sync_copy(zbuf, acc.at[pl.ds(abase + zrep * 16, 16)])
        plsc.subcore_barrier()

        def batch_body(b, _, c_lo=c_lo):
            ebase = sid * _ET + b * _BE
            pltpu.sync_copy(eidx_hbm.at[0, pl.ds(ebase, _BE)], cbat)
            pltpu.sync_copy(eidx_hbm.at[1, pl.ds(ebase, _BE)], nbat)

            def comp_body(g, npend):
                c16 = cbat[pl.ds(g * 16, 16)]
                n16 = nbat[pl.ds(g * 16, 16)]
                msk = (c16 >= c_lo) & (c16 < c_lo + _C)
                mi = msk.astype(jnp.int32)
                offs = npend + plsc.cumsum(mi) - 1
                plsc.store_scatter(pend_c, [offs], c16, mask=msk)
                plsc.store_scatter(pend_n, [offs], n16, mask=msk)
                return npend + jnp.sum(mi)
            npend = lax.fori_loop(0, _BE // 16, comp_body, jnp.int32(0))

            nblk = (npend + (_BR - 1)) // _BR

            def blk_body(blk, _, c_lo=c_lo, npend=npend):
                bb = blk * _BR
                gn = pltpu.async_copy(tbl.at[pend_n.at[pl.ds(bb, _BR)]],
                                      nrow, sem1)
                gc = pltpu.async_copy(tbl.at[pend_c.at[pl.ds(bb, _BR)]],
                                      crow, sem2)
                gn.wait()
                gc.wait()
                nq = jnp.minimum((npend - bb + 127) // 128, _BR // 128)

                def quad_body(q, _, c_lo=c_lo, npend=npend, bb=bb):
                    descs = []
                    for j in range(8):
                        gb = q * 128 + j * 16
                        rem = npend - bb - gb
                        valid = iota < rem
                        rid = gb + iota
                        xn = plsc.load_gather(nrow, [rid, zi])
                        yn = plsc.load_gather(nrow, [rid, zi + 1])
                        zn = plsc.load_gather(nrow, [rid, zi + 2])
                        sv = plsc.load_gather(nrow, [rid, zi + 3])
                        xc = plsc.load_gather(crow, [rid, zi])
                        yc = plsc.load_gather(crow, [rid, zi + 1])
                        zc = plsc.load_gather(crow, [rid, zi + 2])
                        cg = pend_c[pl.ds(bb + gb, 16)]

                        dx = xn - xc
                        dy = yn - yc
                        dz = zn - zc
                        rr = dx * dx + dy * dy + dz * dz + jnp.float32(1e-12)
                        ii = plsc.bitcast(rr, jnp.int32)
                        yv = plsc.bitcast(jnp.int32(0x5F3759DF) - (ii >> 1),
                                          jnp.float32)
                        for _i in range(3):
                            yv = yv * (jnp.float32(1.5)
                                       - jnp.float32(0.5) * rr * yv * yv)
                        rinv = yv
                        r = rr * rinv
                        x = dx * rinv
                        y = dy * rinv
                        z = dz * rinv

                        t = r * jnp.float32(1.0 / _RCUT)
                        u = t - jnp.float32(0.5)
                        w = u * u
                        sp = _poly(w, _CC)
                        sm = u * _poly(w, _CS)
                        cospit = -sm
                        s2 = jnp.float32(2.0) * sp * cospit
                        s3 = sp * (jnp.float32(3.0)
                                   - jnp.float32(4.0) * sp * sp)
                        ok = valid & (r < jnp.float32(_RCUT))
                        pref = jnp.where(
                            ok,
                            jnp.float32(_PREF) * (jnp.float32(1.0) + cospit)
                            * rinv, jnp.float32(0.0))
                        rb = (pref * sp, pref * s2, pref * s3)

                        xx = x * x
                        yy = y * y
                        zz = z * z
                        sh = (
                            jnp.full((16,), 0.28209479177387814, jnp.float32),
                            jnp.float32(0.4886025119029199) * y,
                            jnp.float32(0.4886025119029199) * z,
                            jnp.float32(0.4886025119029199) * x,
                            jnp.float32(1.0925484305920792) * x * y,
                            jnp.float32(1.0925484305920792) * y * z,
                            jnp.float32(0.31539156525252005)
                            * (jnp.float32(3.0) * zz - jnp.float32(1.0)),
                            jnp.float32(1.0925484305920792) * x * z,
                            jnp.float32(0.5462742152960396) * (xx - yy),
                            jnp.float32(0.5900435899266435) * y
                            * (jnp.float32(3.0) * xx - yy),
                            jnp.float32(2.890611442640554) * x * y * z,
                            jnp.float32(0.4570457994644658) * y
                            * (jnp.float32(5.0) * zz - jnp.float32(1.0)),
                            jnp.float32(0.3731763325901154) * z
                            * (jnp.float32(5.0) * zz - jnp.float32(3.0)),
                            jnp.float32(0.4570457994644658) * x
                            * (jnp.float32(5.0) * zz - jnp.float32(1.0)),
                            jnp.float32(1.445305721320277) * z * (xx - yy),
                            jnp.float32(0.5900435899266435) * x
                            * (xx - jnp.float32(3.0) * yy),
                        )

                        vq = vxbufs[j]
                        for mdx in range(16):
                            for kdx in range(3):
                                v = sh[mdx] * rb[kdx]
                                plsc.store_scatter(
                                    vq, [iota, zi + (mdx * 3 + kdx)], v)

                        si = sv.astype(jnp.int32)
                        tgt = jnp.where(valid, (cg - c_lo) * 4 + si,
                                        _DUMMY + (iota & 7))
                        descs.append(pltpu.async_copy(vq, acc.at[tgt], sem3,
                                                      add=True))
                    for d in descs:
                        d.wait()
                    return 0
                lax.fori_loop(0, nq, quad_body, 0)
                return 0
            lax.fori_loop(0, nblk, blk_body, 0)
            return 0
        lax.fori_loop(0, _NB, batch_body, 0)

        plsc.subcore_barrier()

        nbase = sid * _DNT
        nblocks = jnp.where(
            sid < _NT - 1, _DNT // 8,
            jnp.where(chunk == _K - 1, (_N - (_K - 1) * _C - 15 * _DNT) // 8,
                      (_C - 15 * _DNT) // 8))

        def drain_body(blk, _, c_lo=c_lo, nbase=nbase):
            node0 = nbase + blk * 8
            pltpu.sync_copy(acc.at[pl.ds(node0 * 4, 32)], tstage)
            for ov in range(6):
                comp = ov * 16 + iota
                mm = comp // 6
                rem6 = comp - mm * 6
                pp = rem6 // 3
                kk = rem6 - pp * 3
                colv = mm * 3 + kk
                pc0 = plsc.load_gather(pstage, [pp])
                pc1 = plsc.load_gather(pstage, [2 + pp])
                pc2 = plsc.load_gather(pstage, [4 + pp])
                pc3 = plsc.load_gather(pstage, [6 + pp])

                def nl_body(nl, _, colv=colv, pc0=pc0, pc1=pc1, pc2=pc2,
                            pc3=pc3, ov=ov):
                    r0 = zi + nl * 4
                    vacc = pc0 * plsc.load_gather(tstage, [r0, colv])
                    vacc = vacc + pc1 * plsc.load_gather(tstage,
                                                         [r0 + 1, colv])
                    vacc = vacc + pc2 * plsc.load_gather(tstage,
                                                         [r0 + 2, colv])
                    vacc = vacc + pc3 * plsc.load_gather(tstage,
                                                         [r0 + 3, colv])
                    plsc.store_scatter(ostage, [zi + nl, ov * 16 + iota],
                                       vacc)
                    return 0
                lax.fori_loop(0, 8, nl_body, 0)
            pltpu.sync_copy(ostage, out_hbm.at[pl.ds(c_lo + node0, 8)])
            return 0
        lax.fori_loop(0, nblocks, drain_body, 0)

        plsc.subcore_barrier()


@jax.jit
def kernel(positions, edge_index, species_idx, pseudo_embed):
    tbl = jnp.concatenate(
        [positions.astype(jnp.float32),
         species_idx.astype(jnp.float32)[:, None],
         jnp.zeros((_N, 4), jnp.float32)], axis=1)
    eidx = edge_index.astype(jnp.int32)
    pe = pseudo_embed.astype(jnp.float32).reshape(8)

    mesh = plsc.VectorSubcoreMesh(core_axis_name="c", subcore_axis_name="s")
    run = pl.kernel(
        _sc_body,
        out_type=jax.ShapeDtypeStruct((_N, 96), jnp.float32),
        mesh=mesh,
        compiler_params=pltpu.CompilerParams(needs_layout_passes=False,
                                             use_tc_tiling_on_sc=False),
        scratch_types=[
            pltpu.VMEM_SHARED((_N, 8), jnp.float32),
            pltpu.VMEM_SHARED((_AROWS, 48), jnp.float32),
            pltpu.VMEM((_BE,), jnp.int32),
            pltpu.VMEM((_BE,), jnp.int32),
            pltpu.VMEM((_PEND,), jnp.int32),
            pltpu.VMEM((_PEND,), jnp.int32),
            pltpu.VMEM((_BR, 8), jnp.float32),
            pltpu.VMEM((_BR, 8), jnp.float32),
            pltpu.VMEM((16, 48), jnp.float32),
            pltpu.VMEM((16, 48), jnp.float32),
            pltpu.VMEM((16, 48), jnp.float32),
            pltpu.VMEM((16, 48), jnp.float32),
            pltpu.VMEM((16, 48), jnp.float32),
            pltpu.VMEM((16, 48), jnp.float32),
            pltpu.VMEM((16, 48), jnp.float32),
            pltpu.VMEM((16, 48), jnp.float32),
            pltpu.VMEM((32, 48), jnp.float32),
            pltpu.VMEM((8, 96), jnp.float32),
            pltpu.VMEM((16, 48), jnp.float32),
            pltpu.VMEM((8,), jnp.float32),
            pltpu.SemaphoreType.DMA,
            pltpu.SemaphoreType.DMA,
            pltpu.SemaphoreType.DMA,
        ],
        name="spherical_expansion_sc",
    )
    out = run(tbl, eidx, pe)
    return out.reshape(_N, 16, 6)


# speedup vs baseline: 1.0164x; 1.0164x over previous
import jax
import jax.numpy as jnp
from jax import lax
from jax.experimental import pallas as pl
from jax.experimental.pallas import tpu as pltpu
from jax.experimental.pallas import tpu_sc as plsc

_N = 50000
_E = 800000
_K = 8
_C = 6256
_NPAD = _K * _C
_AROWS = 25088
_DUMMY = 4 * _C
_NT = 16
_ET = _E // _NT
_BE = 2000
_NB = _ET // _BE
_PEND = 2048
_BR = 256
_RPT = _AROWS // _NT
_DNT = 392

_NORM = 0.25
_RCUT = 5.0
_PREF = _NORM * 0.6324555320336759 * 0.5

_CC = (0.9999999998456892, -4.934802152338345, 4.058709712157277,
       -1.3352188829542484, 0.2349718054358449, -0.024456210213263066)
_CS = (3.141592653552295, -5.1677127683361075, 2.5501634533581243,
       -0.5992538712565725, 0.08205879138675946, -0.007042952766296561)


def _poly(w, coeffs):
    acc = jnp.float32(coeffs[-1])
    for c in reversed(coeffs[:-1]):
        acc = acc * w + jnp.float32(c)
    return acc


def _sc_body(tbl_hbm, eidx_hbm, pe_hbm, out_hbm,
             tbl, acc, cbat, nbat, pend_c, pend_n,
             crow, nrow, vx0, vx1, vx2, vx3, tstage, ostage, zbuf, pstage,
             sem1, sem2, sem3):
    cid = lax.axis_index("c")
    sid = lax.axis_index("s")
    iota = lax.iota(jnp.int32, 16)
    zf = jnp.zeros((16,), jnp.float32)
    zi = jnp.zeros((16,), jnp.int32)
    vxbufs = (vx0, vx1, vx2, vx3)

    tb = sid * 3128

    @pl.when(sid < _NT - 1)
    def _stage_main():
        pltpu.sync_copy(tbl_hbm.at[pl.ds(tb, 3128)], tbl.at[pl.ds(tb, 3128)])

    @pl.when(sid == _NT - 1)
    def _stage_tail():
        pltpu.sync_copy(tbl_hbm.at[pl.ds(15 * 3128, _N - 15 * 3128)],
                        tbl.at[pl.ds(15 * 3128, _N - 15 * 3128)])

    pltpu.sync_copy(pe_hbm, pstage)

    def zb_body(k, _):
        plsc.store_scatter(zbuf, [jnp.full((16,), k, jnp.int32), iota], zf)
        plsc.store_scatter(zbuf, [jnp.full((16,), k, jnp.int32), 16 + iota],
                           zf)
        plsc.store_scatter(zbuf, [jnp.full((16,), k, jnp.int32), 32 + iota],
                           zf)
        return 0
    lax.fori_loop(0, 16, zb_body, 0)

    def zp_body(k, _):
        plsc.store_scatter(pend_c, [k * 16 + iota], zi)
        plsc.store_scatter(pend_n, [k * 16 + iota], zi)
        return 0
    lax.fori_loop(0, _PEND // 16, zp_body, 0)

    for pidx in range(_K // 2):
        chunk = cid * (_K // 2) + pidx
        c_lo = chunk * _C
        abase = sid * _RPT

        for zrep in range(_RPT // 16):
            pltpu.sync_copy(zbuf, acc.at[pl.ds(abase + zrep * 16, 16)])
        plsc.subcore_barrier()

        def batch_body(b, _, c_lo=c_lo):
            ebase = sid * _ET + b * _BE
            pltpu.sync_copy(eidx_hbm.at[0, pl.ds(ebase, _BE)], cbat)
            pltpu.sync_copy(eidx_hbm.at[1, pl.ds(ebase, _BE)], nbat)

            def comp_body(g, npend):
                c16 = cbat[pl.ds(g * 16, 16)]
                n16 = nbat[pl.ds(g * 16, 16)]
                msk = (c16 >= c_lo) & (c16 < c_lo + _C)
                mi = msk.astype(jnp.int32)
                offs = npend + plsc.cumsum(mi) - 1
                plsc.store_scatter(pend_c, [offs], c16, mask=msk)
                plsc.store_scatter(pend_n, [offs], n16, mask=msk)
                return npend + jnp.sum(mi)
            npend = lax.fori_loop(0, _BE // 16, comp_body, jnp.int32(0))

            nblk = (npend + (_BR - 1)) // _BR

            def blk_body(blk, _, c_lo=c_lo, npend=npend):
                bb = blk * _BR
                gn = pltpu.async_copy(tbl.at[pend_n.at[pl.ds(bb, _BR)]],
                                      nrow, sem1)
                gc = pltpu.async_copy(tbl.at[pend_c.at[pl.ds(bb, _BR)]],
                                      crow, sem2)
                gn.wait()
                gc.wait()
                nq = jnp.minimum((npend - bb + 63) // 64, _BR // 64)

                def quad_body(q, _, c_lo=c_lo, npend=npend, bb=bb):
                    descs = []
                    for j in range(4):
                        gb = q * 64 + j * 16
                        rem = npend - bb - gb
                        valid = iota < rem
                        rid = gb + iota
                        xn = plsc.load_gather(nrow, [rid, zi])
                        yn = plsc.load_gather(nrow, [rid, zi + 1])
                        zn = plsc.load_gather(nrow, [rid, zi + 2])
                        sv = plsc.load_gather(nrow, [rid, zi + 3])
                        xc = plsc.load_gather(crow, [rid, zi])
                        yc = plsc.load_gather(crow, [rid, zi + 1])
                        zc = plsc.load_gather(crow, [rid, zi + 2])
                        cg = pend_c[pl.ds(bb + gb, 16)]

                        dx = xn - xc
                        dy = yn - yc
                        dz = zn - zc
                        rr = dx * dx + dy * dy + dz * dz + jnp.float32(1e-12)
                        ii = plsc.bitcast(rr, jnp.int32)
                        yv = plsc.bitcast(jnp.int32(0x5F3759DF) - (ii >> 1),
                                          jnp.float32)
                        for _i in range(3):
                            yv = yv * (jnp.float32(1.5)
                                       - jnp.float32(0.5) * rr * yv * yv)
                        rinv = yv
                        r = rr * rinv
                        x = dx * rinv
                        y = dy * rinv
                        z = dz * rinv

                        t = r * jnp.float32(1.0 / _RCUT)
                        u = t - jnp.float32(0.5)
                        w = u * u
                        sp = _poly(w, _CC)
                        sm = u * _poly(w, _CS)
                        cospit = -sm
                        s2 = jnp.float32(2.0) * sp * cospit
                        s3 = sp * (jnp.float32(3.0)
                                   - jnp.float32(4.0) * sp * sp)
                        ok = valid & (r < jnp.float32(_RCUT))
                        pref = jnp.where(
                            ok,
                            jnp.float32(_PREF) * (jnp.float32(1.0) + cospit)
                            * rinv, jnp.float32(0.0))
                        rb = (pref * sp, pref * s2, pref * s3)

                        xx = x * x
                        yy = y * y
                        zz = z * z
                        sh = (
                            jnp.full((16,), 0.28209479177387814, jnp.float32),
                            jnp.float32(0.4886025119029199) * y,
                            jnp.float32(0.4886025119029199) * z,
                            jnp.float32(0.4886025119029199) * x,
                            jnp.float32(1.0925484305920792) * x * y,
                            jnp.float32(1.0925484305920792) * y * z,
                            jnp.float32(0.31539156525252005)
                            * (jnp.float32(3.0) * zz - jnp.float32(1.0)),
                            jnp.float32(1.0925484305920792) * x * z,
                            jnp.float32(0.5462742152960396) * (xx - yy),
                            jnp.float32(0.5900435899266435) * y
                            * (jnp.float32(3.0) * xx - yy),
                            jnp.float32(2.890611442640554) * x * y * z,
                            jnp.float32(0.4570457994644658) * y
                            * (jnp.float32(5.0) * zz - jnp.float32(1.0)),
                            jnp.float32(0.3731763325901154) * z
                            * (jnp.float32(5.0) * zz - jnp.float32(3.0)),
                            jnp.float32(0.4570457994644658) * x
                            * (jnp.float32(5.0) * zz - jnp.float32(1.0)),
                            jnp.float32(1.445305721320277) * z * (xx - yy),
                            jnp.float32(0.5900435899266435) * x
                            * (xx - jnp.float32(3.0) * yy),
                        )

                        vq = vxbufs[j]
                        for mdx in range(16):
                            for kdx in range(3):
                                v = sh[mdx] * rb[kdx]
                                plsc.store_scatter(
                                    vq, [iota, zi + (mdx * 3 + kdx)], v)

                        si = sv.astype(jnp.int32)
                        tgt = jnp.where(valid, (cg - c_lo) * 4 + si,
                                        _DUMMY + (iota & 7))
                        descs.append(pltpu.async_copy(vq, acc.at[tgt], sem3,
                                                      add=True))
                    for d in descs:
                        d.wait()
                    return 0
                lax.fori_loop(0, nq, quad_body, 0)
                return 0
            lax.fori_loop(0, nblk, blk_body, 0)
            return 0
        lax.fori_loop(0, _NB, batch_body, 0)

        plsc.subcore_barrier()

        nbase = sid * _DNT
        nblocks = jnp.where(
            sid < _NT - 1, _DNT // 8,
            jnp.where(chunk == _K - 1, (_N - (_K - 1) * _C - 15 * _DNT) // 8,
                      (_C - 15 * _DNT) // 8))

        def drain_body(blk, _, c_lo=c_lo, nbase=nbase):
            node0 = nbase + blk * 8
            pltpu.sync_copy(acc.at[pl.ds(node0 * 4, 32)], tstage)
            for ov in range(6):
                comp = ov * 16 + iota
                mm = comp // 6
                rem6 = comp - mm * 6
                pp = rem6 // 3
                kk = rem6 - pp * 3
                colv = mm * 3 + kk
                pc0 = plsc.load_gather(pstage, [pp])
                pc1 = plsc.load_gather(pstage, [2 + pp])
                pc2 = plsc.load_gather(pstage, [4 + pp])
                pc3 = plsc.load_gather(pstage, [6 + pp])

                def nl_body(nl, _, colv=colv, pc0=pc0, pc1=pc1, pc2=pc2,
                            pc3=pc3, ov=ov):
                    r0 = zi + nl * 4
                    vacc = pc0 * plsc.load_gather(tstage, [r0, colv])
                    vacc = vacc + pc1 * plsc.load_gather(tstage,
                                                         [r0 + 1, colv])
                    vacc = vacc + pc2 * plsc.load_gather(tstage,
                                                         [r0 + 2, colv])
                    vacc = vacc + pc3 * plsc.load_gather(tstage,
                                                         [r0 + 3, colv])
                    plsc.store_scatter(ostage, [zi + nl, ov * 16 + iota],
                                       vacc)
                    return 0
                lax.fori_loop(0, 8, nl_body, 0)
            pltpu.sync_copy(ostage, out_hbm.at[pl.ds(c_lo + node0, 8)])
            return 0
        lax.fori_loop(0, nblocks, drain_body, 0)

        plsc.subcore_barrier()


@jax.jit
def kernel(positions, edge_index, species_idx, pseudo_embed):
    tbl = jnp.concatenate(
        [positions.astype(jnp.float32),
         species_idx.astype(jnp.float32)[:, None],
         jnp.zeros((_N, 4), jnp.float32)], axis=1)
    eidx = edge_index.astype(jnp.int32)
    pe = pseudo_embed.astype(jnp.float32).reshape(8)

    mesh = plsc.VectorSubcoreMesh(core_axis_name="c", subcore_axis_name="s")
    run = pl.kernel(
        _sc_body,
        out_type=jax.ShapeDtypeStruct((_N, 96), jnp.float32),
        mesh=mesh,
        compiler_params=pltpu.CompilerParams(needs_layout_passes=False,
                                             use_tc_tiling_on_sc=False),
        scratch_types=[
            pltpu.VMEM_SHARED((_N, 8), jnp.float32),
            pltpu.VMEM_SHARED((_AROWS, 48), jnp.float32),
            pltpu.VMEM((_BE,), jnp.int32),
            pltpu.VMEM((_BE,), jnp.int32),
            pltpu.VMEM((_PEND,), jnp.int32),
            pltpu.VMEM((_PEND,), jnp.int32),
            pltpu.VMEM((_BR, 8), jnp.float32),
            pltpu.VMEM((_BR, 8), jnp.float32),
            pltpu.VMEM((16, 48), jnp.float32),
            pltpu.VMEM((16, 48), jnp.float32),
            pltpu.VMEM((16, 48), jnp.float32),
            pltpu.VMEM((16, 48), jnp.float32),
            pltpu.VMEM((32, 48), jnp.float32),
            pltpu.VMEM((8, 96), jnp.float32),
            pltpu.VMEM((16, 48), jnp.float32),
            pltpu.VMEM((8,), jnp.float32),
            pltpu.SemaphoreType.DMA,
            pltpu.SemaphoreType.DMA,
            pltpu.SemaphoreType.DMA,
        ],
        name="spherical_expansion_sc",
    )
    out = run(tbl, eidx, pe)
    return out.reshape(_N, 16, 6)
